# baseline (speedup 1.0000x reference)
import functools

import jax
import jax.numpy as jnp
from jax import lax
from jax.experimental import pallas as pl
from jax.experimental.pallas import tpu as pltpu
from jax.experimental.pallas import tpu_sc as plsc

_CACHE = 32768
_SEQ = 2048
_NW = 32
_L = 16
_C_W = _CACHE // _NW
_I_W = _SEQ // _NW

_mesh = plsc.VectorSubcoreMesh(core_axis_name="c", subcore_axis_name="s")


@functools.partial(
    pl.kernel,
    out_type=(
        jax.ShapeDtypeStruct((_SEQ,), jnp.uint32),
        jax.ShapeDtypeStruct((_CACHE,), jnp.uint32),
        jax.ShapeDtypeStruct((_CACHE,), jnp.uint32),
    ),
    mesh=_mesh,
    scratch_types=(
        pltpu.VMEM((_L,), jnp.uint32),
        pltpu.VMEM((_C_W,), jnp.uint32),
        pltpu.VMEM((_C_W,), jnp.uint32),
        pltpu.VMEM((_C_W,), jnp.uint32),
        pltpu.VMEM((_C_W,), jnp.uint32),
        pltpu.VMEM((_I_W,), jnp.uint32),
        pltpu.SemaphoreType.DMA,
        pltpu.SemaphoreType.DMA,
    ),
)
def _sc_update(pos_hbm, seq_hbm, lo_hbm, hi_hbm, idx_hbm, olo_hbm, ohi_hbm,
               pv, lo_v, hi_v, olo_v, ohi_v, idx_v, in_sem, out_sem):
    wid = lax.axis_index("s") * 2 + lax.axis_index("c")
    base = wid * _C_W

    lo_dma = pltpu.async_copy(lo_hbm.at[pl.ds(base, _C_W)], lo_v, in_sem)
    hi_dma = pltpu.async_copy(hi_hbm.at[pl.ds(base, _C_W)], hi_v, in_sem)
    pltpu.sync_copy(pos_hbm, pv.at[pl.ds(0, 1)])
    pltpu.sync_copy(seq_hbm, pv.at[pl.ds(8, 1)])

    par = pv[pl.ds(0, _L)]
    sp = par[0]
    se = sp + par[8] - jnp.uint32(_SEQ)
    lane = lax.iota(jnp.uint32, _L)
    neg1 = jnp.full((_L,), 0xFFFFFFFF, jnp.uint32)
    mask = jnp.uint32(_CACHE - 1)

    ibase = wid * _I_W
    ik = se + jnp.uint32(ibase) + lane

    @plsc.parallel_loop(jnp.int32(0), jnp.int32(_I_W // _L), jnp.int32(1), unroll=4)
    def ibody(j):
        s = j * _L
        idx_v[pl.ds(s, _L)] = (ik + s.astype(jnp.uint32)) & mask

    idx_dma = pltpu.async_copy(idx_v, idx_hbm.at[pl.ds(ibase, _I_W)], out_sem)

    p0 = jnp.uint32(base) + lane
    lo_dma.wait()
    hi_dma.wait()

    @plsc.parallel_loop(jnp.int32(0), jnp.int32(_C_W // _L), jnp.int32(1), unroll=8)
    def body(j):
        s = j * _L
        p = p0 + s.astype(jnp.uint32)
        d = (p - se) & mask
        v = se + d
        win = d < _SEQ
        keep = p < sp
        olo_v[pl.ds(s, _L)] = jnp.where(
            win, v, jnp.where(keep, lo_v[pl.ds(s, _L)], neg1))
        ohi_v[pl.ds(s, _L)] = jnp.where(
            win, jnp.uint32(0) - (v >> 31),
            jnp.where(keep, hi_v[pl.ds(s, _L)], neg1))

    olo_dma = pltpu.async_copy(olo_v, olo_hbm.at[pl.ds(base, _C_W)], out_sem)
    ohi_dma = pltpu.async_copy(ohi_v, ohi_hbm.at[pl.ds(base, _C_W)], out_sem)
    idx_dma.wait()
    olo_dma.wait()
    ohi_dma.wait()


def kernel(input_pos, seq_len, cache_positions):
    pos_lo = input_pos.astype(jnp.uint32)
    seq_lo = jnp.asarray(seq_len, jnp.int64).astype(jnp.uint32).reshape(1)
    lo = cache_positions.astype(jnp.uint32)
    hi = lax.shift_right_logical(cache_positions, 32).astype(jnp.uint32)
    idx_lo, olo, ohi = _sc_update(pos_lo, seq_lo, lo, hi)
    indices = idx_lo.astype(jnp.int64)
    new_cache = (ohi.astype(jnp.int64) << 32) | olo.astype(jnp.int64)
    return indices, new_cache
